# baseline (speedup 1.0000x reference)
import jax
import jax.numpy as jnp
from jax.experimental import pallas as pl
from jax.experimental.pallas import tpu as pltpu

_P = 16
_K = 64
_N = 2048
_D = 512
_PD = _P * _D
_BLK = 512
_CBLK = 256
_NSTRIP = _N // _BLK
_NJ = _N // _CBLK
_R = _BLK // _CBLK
_ITERS = 32 // _NJ

_HI_KEY = 0x3FC00000
_LO_KEY = -0x3FC00001 - 1


def _unmap(m):
    b = jnp.where(m >= 0, m, m ^ 0x7FFFFFFF)
    return jax.lax.bitcast_convert_type(b, jnp.float32)


def _norms_body(ctx_ref, w_ref, inv_ref):
    c = ctx_ref[...]
    w = w_ref[...]
    n2 = jax.lax.dot_general(
        c * c, w * w, (((1,), (1,)), ((), ())),
        preferred_element_type=jnp.float32,
        precision=jax.lax.Precision.HIGHEST)
    inv_ref[...] = 1.0 / jnp.maximum(jnp.sqrt(n2), 1e-12)


def _matmul_block(ctx_ref, w, inv_ref, vr_ref, vc_ref, acc_ref, mir_ref,
                  i, j):
    cj = ctx_ref[pl.ds(j * _CBLK, _CBLK), :]
    invj = inv_ref[pl.ds(j * _CBLK, _CBLK), :]
    for p in range(_P):
        vc_ref[:, p * _D:(p + 1) * _D] = (
            cj * w[p][None, :] * invj[:, p][:, None])

    part = jax.lax.dot_general(
        vr_ref[...], vc_ref[...], (((1,), (1,)), ((), ())),
        preferred_element_type=jnp.float32,
        precision=jax.lax.Precision.DEFAULT) * (1.0 / _P)
    acc_ref[:, pl.ds(j * _CBLK, _CBLK)] = part

    @pl.when(j >= _R * (i + 1))
    def _stash_mirror():
        mir_ref[pl.ds(j * _CBLK - _BLK, _CBLK),
                pl.ds(i * _BLK, _BLK)] = jnp.transpose(part)


def _copy_block(acc_ref, mir_ref, i, j):
    im = jnp.minimum(i, _NSTRIP - 1)
    acc_ref[:, pl.ds(j * _CBLK, _CBLK)] = (
        mir_ref[pl.ds(im * _BLK - _BLK, _BLK), pl.ds(j * _CBLK, _CBLK)])


def _search_chunk(att_ref, lo_ref, hi_ref, out_ref, j):
    @pl.when(j == 0)
    def _init_bounds():
        lo_ref[...] = jnp.full((_BLK, 1), _LO_KEY, jnp.int32)
        hi_ref[...] = jnp.full((_BLK, 1), _HI_KEY, jnp.int32)

    nslab = 4
    sw = _N // nslab

    def step(_, lh):
        lo, hi = lh
        mid = (lo + hi) >> 1
        t = _unmap(mid)
        cnt = jnp.zeros((_BLK, 1), jnp.float32)
        for s in range(nslab):
            slab = att_ref[:, s * sw:(s + 1) * sw]
            cnt = cnt + jnp.sum((slab > t).astype(jnp.float32), axis=1,
                                keepdims=True)
        ge = cnt >= float(_K)
        return jnp.where(ge, mid, lo), jnp.where(ge, hi, mid)

    lo, hi = jax.lax.fori_loop(0, _ITERS, step, (lo_ref[...], hi_ref[...]))
    lo_ref[...] = lo
    hi_ref[...] = hi

    @pl.when(j == _NJ - 1)
    def _mask_and_write():
        t = _unmap(lo)
        for s in range(nslab):
            slab = att_ref[:, s * sw:(s + 1) * sw]
            out_ref[:, s * sw:(s + 1) * sw] = jnp.where(slab > t, slab, 0.0)


def _body(ctx_ref, w_ref, inv_ref, out_ref, vr_ref, vc_ref, mir_ref,
          acc0_ref, acc1_ref, lo_ref, hi_ref):
    i = pl.program_id(0)
    j = pl.program_id(1)
    w = w_ref[...]

    @pl.when(jnp.logical_and(i < _NSTRIP, j == 0))
    def _build_rows():
        c = ctx_ref[pl.ds(i * _BLK, _BLK), :]
        inv = inv_ref[pl.ds(i * _BLK, _BLK), :]
        for p in range(_P):
            vr_ref[:, p * _D:(p + 1) * _D] = (
                c * w[p][None, :] * inv[:, p][:, None])

    even = jax.lax.rem(i, 2) == 0
    upper = j >= _R * i

    @pl.when(jnp.logical_and(upper, even))
    def _a():
        _matmul_block(ctx_ref, w, inv_ref, vr_ref, vc_ref, acc0_ref,
                      mir_ref, i, j)
        _search_chunk(acc1_ref, lo_ref, hi_ref, out_ref, j)

    @pl.when(jnp.logical_and(upper, jnp.logical_not(even)))
    def _b():
        _matmul_block(ctx_ref, w, inv_ref, vr_ref, vc_ref, acc1_ref,
                      mir_ref, i, j)
        _search_chunk(acc0_ref, lo_ref, hi_ref, out_ref, j)

    @pl.when(jnp.logical_and(jnp.logical_not(upper), even))
    def _c():
        _copy_block(acc0_ref, mir_ref, i, j)
        _search_chunk(acc1_ref, lo_ref, hi_ref, out_ref, j)

    @pl.when(jnp.logical_and(jnp.logical_not(upper), jnp.logical_not(even)))
    def _d():
        _copy_block(acc1_ref, mir_ref, i, j)
        _search_chunk(acc0_ref, lo_ref, hi_ref, out_ref, j)


@jax.jit
def kernel(context, W):
    inv = pl.pallas_call(
        _norms_body,
        grid=(_NSTRIP,),
        in_specs=[
            pl.BlockSpec((_BLK, _D), lambda i: (i, 0)),
            pl.BlockSpec((_P, _D), lambda i: (0, 0)),
        ],
        out_specs=pl.BlockSpec((_BLK, _P), lambda i: (i, 0)),
        out_shape=jax.ShapeDtypeStruct((_N, _P), jnp.float32),
    )(context, W)

    return pl.pallas_call(
        _body,
        grid=(_NSTRIP + 1, _NJ),
        in_specs=[
            pl.BlockSpec((_N, _D), lambda i, j: (0, 0)),
            pl.BlockSpec((_P, _D), lambda i, j: (0, 0)),
            pl.BlockSpec((_N, _P), lambda i, j: (0, 0)),
        ],
        out_specs=pl.BlockSpec((_BLK, _N),
                               lambda i, j: (jnp.maximum(i, 1) - 1, 0)),
        out_shape=jax.ShapeDtypeStruct((_N, _N), jnp.float32),
        scratch_shapes=[
            pltpu.VMEM((_BLK, _PD), jnp.float32),
            pltpu.VMEM((_CBLK, _PD), jnp.float32),
            pltpu.VMEM((_N - _BLK, _N - _BLK), jnp.float32),
            pltpu.VMEM((_BLK, _N), jnp.float32),
            pltpu.VMEM((_BLK, _N), jnp.float32),
            pltpu.VMEM((_BLK, 1), jnp.int32),
            pltpu.VMEM((_BLK, 1), jnp.int32),
        ],
    )(context, W, inv)


# speedup vs baseline: 1.2215x; 1.2215x over previous
import jax
import jax.numpy as jnp
from jax.experimental import pallas as pl
from jax.experimental.pallas import tpu as pltpu

_P = 16
_K = 64
_N = 2048
_D = 512
_PD = _P * _D
_BLK = 512
_CBLK = 512
_NSTRIP = _N // _BLK
_NJ = _N // _CBLK

_HI_KEY = 0x3FC00000
_LO_KEY = -0x3FC00001 - 1


def _unmap(m):
    b = jnp.where(m >= 0, m, m ^ 0x7FFFFFFF)
    return jax.lax.bitcast_convert_type(b, jnp.float32)


def _norms_body(ctx_ref, w_ref, inv_ref):
    c = ctx_ref[...]
    w = w_ref[...]
    n2 = jax.lax.dot_general(
        c * c, w * w, (((1,), (1,)), ((), ())),
        preferred_element_type=jnp.float32,
        precision=jax.lax.Precision.HIGHEST)
    inv_ref[...] = 1.0 / jnp.maximum(jnp.sqrt(n2), 1e-12)


def _body(ctx_ref, w_ref, inv_ref, out_ref, vr_ref, vc_ref, mir_ref):
    i = pl.program_id(0)
    j = pl.program_id(1)
    w = w_ref[...]

    @pl.when(j == 0)
    def _build_rows():
        c = ctx_ref[pl.ds(i * _BLK, _BLK), :]
        inv = inv_ref[pl.ds(i * _BLK, _BLK), :]
        for p in range(_P):
            vr_ref[:, p * _D:(p + 1) * _D] = (
                c * w[p][None, :] * inv[:, p][:, None])

    @pl.when(j >= i)
    def _upper():
        cj = ctx_ref[pl.ds(j * _CBLK, _CBLK), :]
        invj = inv_ref[pl.ds(j * _CBLK, _CBLK), :]
        for p in range(_P):
            vc_ref[:, p * _D:(p + 1) * _D] = (
                cj * w[p][None, :] * invj[:, p][:, None])

        part = jax.lax.dot_general(
            vr_ref[...], vc_ref[...], (((1,), (1,)), ((), ())),
            preferred_element_type=jnp.float32,
            precision=jax.lax.Precision.DEFAULT) * (1.0 / _P)
        out_ref[:, pl.ds(j * _CBLK, _CBLK)] = part

        @pl.when(j >= i + 1)
        def _stash_mirror():
            mir_ref[pl.ds(j * _CBLK - _BLK, _CBLK),
                    pl.ds(i * _BLK, _BLK)] = jnp.transpose(part)

    @pl.when(j < i)
    def _copy_mirror():
        out_ref[:, pl.ds(j * _CBLK, _CBLK)] = (
            mir_ref[pl.ds(i * _BLK - _BLK, _BLK), pl.ds(j * _CBLK, _CBLK)])

    @pl.when(j == _NJ - 1)
    def _epilogue():
        nslab = 4
        sw = _N // nslab
        lo0 = jnp.full((_BLK, 1), _LO_KEY, jnp.int32)
        hi0 = jnp.full((_BLK, 1), _HI_KEY, jnp.int32)

        def step(_, lh):
            lo, hi = lh
            mid = (lo + hi) >> 1
            t = _unmap(mid)
            cnt = jnp.zeros((_BLK, 1), jnp.float32)
            for s in range(nslab):
                slab = out_ref[:, s * sw:(s + 1) * sw]
                cnt = cnt + jnp.sum((slab > t).astype(jnp.float32), axis=1,
                                    keepdims=True)
            ge = cnt >= float(_K)
            return jnp.where(ge, mid, lo), jnp.where(ge, hi, mid)

        lo, _ = jax.lax.fori_loop(0, 31, step, (lo0, hi0))
        t = _unmap(lo)
        for s in range(nslab):
            slab = out_ref[:, s * sw:(s + 1) * sw]
            out_ref[:, s * sw:(s + 1) * sw] = jnp.where(slab > t, slab, 0.0)


@jax.jit
def kernel(context, W):
    inv = pl.pallas_call(
        _norms_body,
        grid=(_NSTRIP,),
        in_specs=[
            pl.BlockSpec((_BLK, _D), lambda i: (i, 0)),
            pl.BlockSpec((_P, _D), lambda i: (0, 0)),
        ],
        out_specs=pl.BlockSpec((_BLK, _P), lambda i: (i, 0)),
        out_shape=jax.ShapeDtypeStruct((_N, _P), jnp.float32),
    )(context, W)

    return pl.pallas_call(
        _body,
        grid=(_NSTRIP, _NJ),
        in_specs=[
            pl.BlockSpec((_N, _D), lambda i, j: (0, 0)),
            pl.BlockSpec((_P, _D), lambda i, j: (0, 0)),
            pl.BlockSpec((_N, _P), lambda i, j: (0, 0)),
        ],
        out_specs=pl.BlockSpec((_BLK, _N), lambda i, j: (i, 0)),
        out_shape=jax.ShapeDtypeStruct((_N, _N), jnp.float32),
        scratch_shapes=[
            pltpu.VMEM((_BLK, _PD), jnp.float32),
            pltpu.VMEM((_CBLK, _PD), jnp.float32),
            pltpu.VMEM((_N - _BLK, _N - _BLK), jnp.float32),
        ],
    )(context, W, inv)
